# baseline (speedup 1.0000x reference)
import jax
import jax.numpy as jnp
from jax import lax
from jax.experimental import pallas as pl
from jax.experimental.pallas import tpu as pltpu
from jax.experimental.pallas import tpu_sc as plsc

N = 16384
D = 128
DW = D // 2
L = 16
NW = 32
PAIRS_PER_W = N // NW
CHUNK = 64
NCHUNK = PAIRS_PER_W // CHUNK
NBUF = 4
GROUPS = CHUNK // L
UNROLL = 8


def _body(x0_hbm, x1_hbm, u_hbm, v_hbm, out_hbm,
          idx0_v, idx1_v, u_buf0, u_buf1, u_buf2, u_buf3,
          v_buf0, v_buf1, v_buf2, v_buf3, out_v, sems):
    nc = 2
    wid = lax.axis_index("s") * nc + lax.axis_index("c")
    base = wid * PAIRS_PER_W

    u_bufs = (u_buf0, u_buf1, u_buf2, u_buf3)
    v_bufs = (v_buf0, v_buf1, v_buf2, v_buf3)

    iota = lax.iota(jnp.int32, L)

    pltpu.sync_copy(x0_hbm.at[pl.ds(base, PAIRS_PER_W)], idx0_v)
    pltpu.sync_copy(x1_hbm.at[pl.ds(base, PAIRS_PER_W)], idx1_v)

    def start(t, slot):
        pltpu.async_copy(u_hbm.at[idx0_v.at[pl.ds(t * CHUNK, CHUNK)]],
                         u_bufs[slot], sems.at[slot, 0])
        pltpu.async_copy(v_hbm.at[idx1_v.at[pl.ds(t * CHUNK, CHUNK)]],
                         v_bufs[slot], sems.at[slot, 1])

    def wait(slot):
        pltpu.make_async_copy(u_hbm.at[idx0_v.at[pl.ds(0, CHUNK)]],
                              u_bufs[slot], sems.at[slot, 0]).wait()
        pltpu.make_async_copy(v_hbm.at[idx1_v.at[pl.ds(0, CHUNK)]],
                              v_bufs[slot], sems.at[slot, 1]).wait()

    for slot in range(NBUF):
        start(slot, slot)

    def chunk_pair(i, _):
        tt = i * NBUF
        for b in range(NBUF):
            ct = tt + b
            wait(b)

            def group(g, _g):
                rows = iota + g * L

                def dot_step(t8, acc):
                    jb = iota + t8 * UNROLL
                    for u in range(UNROLL):
                        js = (jb + u) & (DW - 1)
                        ui = plsc.load_gather(u_bufs[b], [rows, js])
                        vi = plsc.load_gather(v_bufs[b], [rows, js])
                        ulo, uhi = plsc.unpack(
                            plsc.bitcast(ui, jnp.bfloat16),
                            format=plsc.PackFormat.INTERLEAVED)
                        vlo, vhi = plsc.unpack(
                            plsc.bitcast(vi, jnp.bfloat16),
                            format=plsc.PackFormat.INTERLEAVED)
                        acc = acc + ulo * vlo + uhi * vhi
                    return acc

                acc = lax.fori_loop(0, DW // UNROLL, dot_step,
                                    jnp.zeros((L,), jnp.float32))
                prob = 1.0 / (1.0 + jnp.exp(-acc))
                out_v[pl.ds(ct * CHUNK + g * L, L)] = prob
                return 0

            lax.fori_loop(0, GROUPS, group, 0)

            nxt = ct + NBUF

            @pl.when(nxt < NCHUNK)
            def _():
                start(nxt, b)
        return 0

    lax.fori_loop(0, NCHUNK // NBUF, chunk_pair, 0)

    pltpu.sync_copy(out_v, out_hbm.at[pl.ds(base, PAIRS_PER_W)])


@jax.jit
def _run(x0, x1, u, v):
    mesh = plsc.VectorSubcoreMesh(core_axis_name="c", subcore_axis_name="s")
    return pl.kernel(
        _body,
        out_type=jax.ShapeDtypeStruct((N,), jnp.float32),
        mesh=mesh,
        scratch_types=[
            pltpu.VMEM((PAIRS_PER_W,), jnp.int32),
            pltpu.VMEM((PAIRS_PER_W,), jnp.int32),
            pltpu.VMEM((CHUNK, DW), jnp.int32),
            pltpu.VMEM((CHUNK, DW), jnp.int32),
            pltpu.VMEM((CHUNK, DW), jnp.int32),
            pltpu.VMEM((CHUNK, DW), jnp.int32),
            pltpu.VMEM((CHUNK, DW), jnp.int32),
            pltpu.VMEM((CHUNK, DW), jnp.int32),
            pltpu.VMEM((CHUNK, DW), jnp.int32),
            pltpu.VMEM((CHUNK, DW), jnp.int32),
            pltpu.VMEM((PAIRS_PER_W,), jnp.float32),
            pltpu.SemaphoreType.DMA((NBUF, 2)),
        ],
        compiler_params=pltpu.CompilerParams(
            needs_layout_passes=False,
            disable_bounds_checks=True,
            skip_device_barrier=True,
            use_tc_tiling_on_sc=False,
        ),
    )(x0, x1, u, v)


def _pack(t):
    b = t.astype(jnp.bfloat16).reshape(t.shape[0], DW, 2)
    return jax.lax.bitcast_convert_type(b, jnp.int32)


def kernel(X, U, V):
    return _run(X[:, 0], X[:, 1], _pack(U), _pack(V))


# speedup vs baseline: 1.0819x; 1.0819x over previous
import jax
import jax.numpy as jnp
from jax import lax
from jax.experimental import pallas as pl
from jax.experimental.pallas import tpu as pltpu
from jax.experimental.pallas import tpu_sc as plsc

N = 16384
D = 128
DW = D // 2
L = 16
NW = 32
PAIRS_PER_W = N // NW
CHUNK = 64
NCHUNK = PAIRS_PER_W // CHUNK
NBUF = 4
GROUPS = CHUNK // L
UNROLL = 8


def _body(x0_hbm, x1_hbm, u_hbm, v_hbm, out_hbm,
          idx0_v, idx1_v, u_buf0, u_buf1, u_buf2, u_buf3,
          v_buf0, v_buf1, v_buf2, v_buf3, out_v, sems):
    nc = 2
    wid = lax.axis_index("s") * nc + lax.axis_index("c")
    base = wid * PAIRS_PER_W

    u_bufs = (u_buf0, u_buf1, u_buf2, u_buf3)
    v_bufs = (v_buf0, v_buf1, v_buf2, v_buf3)

    iota = lax.iota(jnp.int32, L)

    pltpu.sync_copy(x0_hbm.at[pl.ds(base, PAIRS_PER_W)], idx0_v)
    pltpu.sync_copy(x1_hbm.at[pl.ds(base, PAIRS_PER_W)], idx1_v)

    def start(t, slot):
        pltpu.async_copy(u_hbm.at[idx0_v.at[pl.ds(t * CHUNK, CHUNK)]],
                         u_bufs[slot], sems.at[slot, 0])
        pltpu.async_copy(v_hbm.at[idx1_v.at[pl.ds(t * CHUNK, CHUNK)]],
                         v_bufs[slot], sems.at[slot, 1])

    def wait(slot):
        pltpu.make_async_copy(u_hbm.at[idx0_v.at[pl.ds(0, CHUNK)]],
                              u_bufs[slot], sems.at[slot, 0]).wait()
        pltpu.make_async_copy(v_hbm.at[idx1_v.at[pl.ds(0, CHUNK)]],
                              v_bufs[slot], sems.at[slot, 1]).wait()

    for slot in range(NBUF):
        start(slot, slot)

    def chunk_pair(i, _):
        tt = i * NBUF
        for b in range(NBUF):
            ct = tt + b
            wait(b)

            def group(g, _g):
                rows = iota + g * L

                def dot_step(t8, acc):
                    jb = iota + t8 * UNROLL
                    for u in range(UNROLL):
                        js = (jb + u) & (DW - 1)
                        ui = plsc.load_gather(u_bufs[b], [rows, js])
                        vi = plsc.load_gather(v_bufs[b], [rows, js])
                        ulo, uhi = plsc.unpack(
                            plsc.bitcast(ui, jnp.bfloat16),
                            format=plsc.PackFormat.INTERLEAVED)
                        vlo, vhi = plsc.unpack(
                            plsc.bitcast(vi, jnp.bfloat16),
                            format=plsc.PackFormat.INTERLEAVED)
                        acc = acc + ulo * vlo + uhi * vhi
                    return acc

                acc = lax.fori_loop(0, DW // UNROLL, dot_step,
                                    jnp.zeros((L,), jnp.float32))
                prob = 1.0 / (1.0 + jnp.exp(-acc))
                out_v[pl.ds(ct * CHUNK + g * L, L)] = prob
                return 0

            lax.fori_loop(0, GROUPS, group, 0)

            nxt = ct + NBUF

            @pl.when(nxt < NCHUNK)
            def _():
                start(nxt, b)
        return 0

    lax.fori_loop(0, NCHUNK // NBUF, chunk_pair, 0)

    pltpu.sync_copy(out_v, out_hbm.at[pl.ds(base, PAIRS_PER_W)])


@jax.jit
def _run(x0, x1, u, v):
    mesh = plsc.VectorSubcoreMesh(core_axis_name="c", subcore_axis_name="s")
    return pl.kernel(
        _body,
        out_type=jax.ShapeDtypeStruct((N,), jnp.float32),
        mesh=mesh,
        scratch_types=[
            pltpu.VMEM((PAIRS_PER_W,), jnp.int32),
            pltpu.VMEM((PAIRS_PER_W,), jnp.int32),
            pltpu.VMEM((CHUNK, DW), jnp.int32),
            pltpu.VMEM((CHUNK, DW), jnp.int32),
            pltpu.VMEM((CHUNK, DW), jnp.int32),
            pltpu.VMEM((CHUNK, DW), jnp.int32),
            pltpu.VMEM((CHUNK, DW), jnp.int32),
            pltpu.VMEM((CHUNK, DW), jnp.int32),
            pltpu.VMEM((CHUNK, DW), jnp.int32),
            pltpu.VMEM((CHUNK, DW), jnp.int32),
            pltpu.VMEM((PAIRS_PER_W,), jnp.float32),
            pltpu.SemaphoreType.DMA((NBUF, 2)),
        ],
        compiler_params=pltpu.CompilerParams(
            needs_layout_passes=False,
            disable_bounds_checks=True,
            skip_device_barrier=True,
            use_tc_tiling_on_sc=False,
        ),
    )(x0, x1, u, v)


def _pack(t):
    b = t.astype(jnp.bfloat16)
    lo = jax.lax.bitcast_convert_type(b[:, :DW], jnp.uint16).astype(jnp.uint32)
    hi = jax.lax.bitcast_convert_type(b[:, DW:], jnp.uint16).astype(jnp.uint32)
    return jax.lax.bitcast_convert_type(lo | (hi << 16), jnp.int32)


def kernel(X, U, V):
    return _run(X[:, 0], X[:, 1], _pack(U), _pack(V))
